# Initial kernel scaffold; baseline (speedup 1.0000x reference)
import jax
import jax.numpy as jnp
from jax.experimental import pallas as pl


def kernel(x, Wg, bg, W1, b1, W2, b2):
    raise NotImplementedError("write your pallas kernel here")


# speedup vs baseline: 1.3204x; 1.3204x over previous
import functools

import jax
import jax.numpy as jnp
from jax.experimental import pallas as pl
from jax.experimental.pallas import tpu as pltpu

E = 64
K = 2
D = 768
F = 768
T = 64
ALPHA = 1.702
BETA = 1.0


def _moe_body(x_ref, wg_ref, bg_ref, w1_ref, b1_ref, w2_ref, b2_ref,
              out_ref, rw_ref):
    e = pl.program_id(0)
    lanes = jax.lax.broadcasted_iota(jnp.int32, (T, E), 1)

    @pl.when(e == 0)
    def _router():
        g = jnp.dot(x_ref[...], wg_ref[...],
                    preferred_element_type=jnp.float32) + bg_ref[...]
        idx1 = jnp.argmax(g, axis=-1)
        m1 = jnp.max(g, axis=-1)
        g2 = jnp.where(lanes == idx1[:, None], -jnp.inf, g)
        idx2 = jnp.argmax(g2, axis=-1)
        m2 = jnp.max(g2, axis=-1)
        z = jnp.exp(m2 - m1)
        p1 = 1.0 / (1.0 + z)
        p2 = z / (1.0 + z)
        rw_ref[...] = (jnp.where(lanes == idx1[:, None], p1[:, None], 0.0)
                       + jnp.where(lanes == idx2[:, None], p2[:, None], 0.0))
        out_ref[...] = jnp.zeros_like(out_ref)

    w = jnp.sum(jnp.where(lanes == e, rw_ref[...], 0.0), axis=1,
                keepdims=True)
    h = jnp.dot(x_ref[...], w1_ref[0],
                preferred_element_type=jnp.float32) + b1_ref[0]
    glu = h[:, :F]
    lin = h[:, F:]
    act = glu * jax.nn.sigmoid(ALPHA * glu) * (lin + BETA)
    o = jnp.dot(act, w2_ref[0],
                preferred_element_type=jnp.float32) + b2_ref[0]
    out_ref[...] += w * o


@jax.jit
def kernel(x, Wg, bg, W1, b1, W2, b2):
    out = pl.pallas_call(
        _moe_body,
        grid=(E,),
        in_specs=[
            pl.BlockSpec((T, D), lambda e: (0, 0)),
            pl.BlockSpec((D, E), lambda e: (0, 0)),
            pl.BlockSpec((1, E), lambda e: (0, 0)),
            pl.BlockSpec((1, D, 2 * F), lambda e: (e, 0, 0)),
            pl.BlockSpec((1, 1, 2 * F), lambda e: (e, 0, 0)),
            pl.BlockSpec((1, F, D), lambda e: (e, 0, 0)),
            pl.BlockSpec((1, 1, D), lambda e: (e, 0, 0)),
        ],
        out_specs=pl.BlockSpec((T, D), lambda e: (0, 0)),
        out_shape=jax.ShapeDtypeStruct((T, D), jnp.float32),
        scratch_shapes=[pltpu.VMEM((T, E), jnp.float32)],
        compiler_params=pltpu.CompilerParams(
            dimension_semantics=("arbitrary",),
        ),
    )(x, Wg, bg.reshape(1, E), W1, b1.reshape(E, 1, 2 * F), W2,
      b2.reshape(E, 1, D))
    return out.reshape(x.shape)


# speedup vs baseline: 1.4554x; 1.1023x over previous
import jax
import jax.numpy as jnp
from jax.experimental import pallas as pl
from jax.experimental.pallas import tpu as pltpu

E = 64
K = 2
D = 768
F = 768
T = 64
ALPHA = 1.702
BETA = 1.0


def _router_body(x_ref, wg_ref, bg_ref, rw_ref, active_ref, nact_ref):
    lanes = jax.lax.broadcasted_iota(jnp.int32, (T, E), 1)
    g = jnp.dot(x_ref[...], wg_ref[...],
                preferred_element_type=jnp.float32) + bg_ref[...]
    idx1 = jnp.argmax(g, axis=-1)
    m1 = jnp.max(g, axis=-1)
    g2 = jnp.where(lanes == idx1[:, None], -jnp.inf, g)
    idx2 = jnp.argmax(g2, axis=-1)
    m2 = jnp.max(g2, axis=-1)
    z = jnp.exp(m2 - m1)
    p1 = 1.0 / (1.0 + z)
    p2 = z / (1.0 + z)
    rw = (jnp.where(lanes == idx1[:, None], p1[:, None], 0.0)
          + jnp.where(lanes == idx2[:, None], p2[:, None], 0.0))
    rw_ref[...] = rw

    hit_row = (jnp.sum(rw, axis=0, keepdims=True) > 0.0)
    hitf = hit_row.astype(jnp.float32)
    r = jax.lax.broadcasted_iota(jnp.int32, (E, E), 0)
    c = jax.lax.broadcasted_iota(jnp.int32, (E, E), 1)
    upper = (r <= c).astype(jnp.float32)
    cum_row = jnp.dot(hitf, upper,
                      preferred_element_type=jnp.float32)
    cum_b = jnp.broadcast_to(cum_row, (E, E))
    slot = jax.lax.broadcasted_iota(jnp.int32, (E, E), 0).astype(jnp.float32)
    sel = jnp.where((cum_b == slot + 1.0) & jnp.broadcast_to(hit_row, (E, E)),
                    1.0, 0.0)
    e_vals = c.astype(jnp.float32)
    active_col = jnp.sum(sel * e_vals, axis=1, keepdims=True)
    active_ref[...] = active_col.astype(jnp.int32)
    nact_ref[...] = jnp.sum(hitf, dtype=jnp.float32).astype(jnp.int32)[
        None, None]


def _expert_body(active_ref, nact_ref, x_ref, rw_ref, w1_ref, b1_ref,
                 w2_ref, b2_ref, out_ref):
    i = pl.program_id(0)
    n = nact_ref[0]

    @pl.when(i == 0)
    def _init():
        out_ref[...] = jnp.zeros_like(out_ref)

    @pl.when(i < n)
    def _compute():
        e = active_ref[jnp.minimum(i, n - 1)]
        lanes = jax.lax.broadcasted_iota(jnp.int32, (T, E), 1)
        w = jnp.sum(jnp.where(lanes == e, rw_ref[...], 0.0), axis=1,
                    keepdims=True)
        h = jnp.dot(x_ref[...], w1_ref[0],
                    preferred_element_type=jnp.float32) + b1_ref[0]
        glu = h[:, :F]
        lin = h[:, F:]
        act = glu * jax.nn.sigmoid(ALPHA * glu) * (lin + BETA)
        o = jnp.dot(act, w2_ref[0],
                    preferred_element_type=jnp.float32) + b2_ref[0]
        out_ref[...] += w * o


@jax.jit
def kernel(x, Wg, bg, W1, b1, W2, b2):
    rw, active, nact = pl.pallas_call(
        _router_body,
        in_specs=[
            pl.BlockSpec((T, D), lambda: (0, 0)),
            pl.BlockSpec((D, E), lambda: (0, 0)),
            pl.BlockSpec((1, E), lambda: (0, 0)),
        ],
        out_specs=[
            pl.BlockSpec((T, E), lambda: (0, 0)),
            pl.BlockSpec((E, 1), lambda: (0, 0)),
            pl.BlockSpec((1, 1), lambda: (0, 0)),
        ],
        out_shape=[
            jax.ShapeDtypeStruct((T, E), jnp.float32),
            jax.ShapeDtypeStruct((E, 1), jnp.int32),
            jax.ShapeDtypeStruct((1, 1), jnp.int32),
        ],
    )(x, Wg, bg.reshape(1, E))

    def clamp(i, a_ref, n_ref):
        return a_ref[jnp.minimum(i, n_ref[0] - 1)]

    out = pl.pallas_call(
        _expert_body,
        grid_spec=pltpu.PrefetchScalarGridSpec(
            num_scalar_prefetch=2,
            grid=(E,),
            in_specs=[
                pl.BlockSpec((T, D), lambda i, a, nn: (0, 0)),
                pl.BlockSpec((T, E), lambda i, a, nn: (0, 0)),
                pl.BlockSpec((1, D, 2 * F),
                             lambda i, a, nn: (clamp(i, a, nn), 0, 0)),
                pl.BlockSpec((1, 1, 2 * F),
                             lambda i, a, nn: (clamp(i, a, nn), 0, 0)),
                pl.BlockSpec((1, F, D),
                             lambda i, a, nn: (clamp(i, a, nn), 0, 0)),
                pl.BlockSpec((1, 1, D),
                             lambda i, a, nn: (clamp(i, a, nn), 0, 0)),
            ],
            out_specs=pl.BlockSpec((T, D), lambda i, a, nn: (0, 0)),
        ),
        out_shape=jax.ShapeDtypeStruct((T, D), jnp.float32),
        compiler_params=pltpu.CompilerParams(
            dimension_semantics=("arbitrary",),
        ),
    )(active.reshape(E), nact.reshape(1), x, rw, W1,
      b1.reshape(E, 1, 2 * F), W2, b2.reshape(E, 1, D))
    return out.reshape(x.shape)
